# Initial kernel scaffold; baseline (speedup 1.0000x reference)
import jax
import jax.numpy as jnp
from jax.experimental import pallas as pl


def kernel(x, w_g, c_fc, c_proj):
    raise NotImplementedError("write your pallas kernel here")


# speedup vs baseline: 1.5666x; 1.5666x over previous
import functools

import jax
import jax.numpy as jnp
from jax import lax
from jax.experimental import pallas as pl
from jax.experimental.pallas import tpu as pltpu
from jax.experimental.pallas import tpu_sc as plsc

D = 768
H = 4 * D
E = 8
TOPK = 2
T = 2048
CAP = 640
S = E * CAP
NENT = TOPK * T
NC = 2
NS = 16
NW = NC * NS

_BLK = 128
_NBLK = NENT // _BLK
_SBLK = S // 128


def _router_body(x_ref, wg_ref, perm_ref, slotc_ref, w_ref, slotd_ref, tok_ref):
    xv = x_ref[...]
    wg = wg_ref[...]
    logits = jnp.dot(xv, wg, preferred_element_type=jnp.float32)

    ii = lax.broadcasted_iota(jnp.int32, (T, E), 1)
    l0 = jnp.max(logits, axis=1, keepdims=True)
    e0 = jnp.min(jnp.where(logits == l0, ii, E), axis=1, keepdims=True)
    m1 = jnp.where(ii == e0, -jnp.inf, logits)
    l1 = jnp.max(m1, axis=1, keepdims=True)
    e1 = jnp.min(jnp.where(m1 == l1, ii, E), axis=1, keepdims=True)
    z = jnp.exp(l1 - l0)
    p0 = 1.0 / (1.0 + z)
    p1 = z / (1.0 + z)

    r = lax.broadcasted_iota(jnp.int32, (_BLK, _BLK), 0)
    c = lax.broadcasted_iota(jnp.int32, (_BLK, _BLK), 1)
    tri = (r >= c).astype(jnp.float32)
    blk_ii = lax.broadcasted_iota(jnp.int32, (_BLK, E), 1)
    row_iota = lax.broadcasted_iota(jnp.int32, (_BLK, 1), 0)

    carry = jnp.zeros((1, E), jnp.float32)
    for b in range(_NBLK):
        if b < _NBLK // 2:
            eb = lax.slice(e0, (b * _BLK, 0), ((b + 1) * _BLK, 1))
            pb = lax.slice(p0, (b * _BLK, 0), ((b + 1) * _BLK, 1))
            tokb = b * _BLK + row_iota
        else:
            b2 = b - _NBLK // 2
            eb = lax.slice(e1, (b2 * _BLK, 0), ((b2 + 1) * _BLK, 1))
            pb = lax.slice(p1, (b2 * _BLK, 0), ((b2 + 1) * _BLK, 1))
            tokb = b2 * _BLK + row_iota
        mb = (blk_ii == eb).astype(jnp.float32)
        inc = jnp.dot(tri, mb, preferred_element_type=jnp.float32) + carry
        carry = carry + jnp.sum(mb, axis=0, keepdims=True)
        rank = jnp.sum(mb * inc, axis=1, keepdims=True).astype(jnp.int32) - 1
        kept = rank < CAP
        slot_d = jnp.where(kept, eb * CAP + rank, -1)
        slot_c = jnp.where(kept, eb * CAP + rank, eb * CAP + (CAP - 1))
        w_out = jnp.where(kept, pb, 0.0)
        sl = pl.ds(b * _BLK, _BLK)
        slotd_ref[sl, :] = slot_d
        slotc_ref[sl, :] = slot_c
        w_ref[sl, :] = jnp.broadcast_to(w_out, (_BLK, 16))
        tok_ref[sl, :] = tokb.astype(jnp.float32)

    lane = lax.broadcasted_iota(jnp.int32, (1, 128), 1)

    def sblk_body(b, _):
        svals = b * 128 + lane

        def chunk(ic, acc):
            sd = slotd_ref[pl.ds(ic * 512, 512), :]
            tk = tok_ref[pl.ds(ic * 512, 512), :]
            m = sd == svals
            return acc + jnp.sum(jnp.where(m, tk, 0.0), axis=0, keepdims=True)

        acc = lax.fori_loop(0, NENT // 512, chunk, jnp.zeros((1, 128), jnp.float32))
        perm_ref[pl.ds(b, 1), :] = acc.astype(jnp.int32)
        return 0

    lax.fori_loop(0, _SBLK, sblk_body, 0)


def _router(x2, w_g):
    return pl.pallas_call(
        _router_body,
        out_shape=[
            jax.ShapeDtypeStruct((_SBLK, 128), jnp.int32),
            jax.ShapeDtypeStruct((NENT, 1), jnp.int32),
            jax.ShapeDtypeStruct((NENT, 16), jnp.float32),
        ],
        scratch_shapes=[
            pltpu.VMEM((NENT, 1), jnp.int32),
            pltpu.VMEM((NENT, 1), jnp.float32),
        ],
    )(x2, w_g)


_ROWS_PER_W = S // NW
_HALF = _ROWS_PER_W // 2


def _dispatch_body(x_hbm, perm_hbm, out_hbm, idx_v, rows0, rows1, sem):
    wid = lax.axis_index("s") * NC + lax.axis_index("c")
    base = wid * _ROWS_PER_W
    pltpu.sync_copy(perm_hbm.at[pl.ds(base, _HALF)], idx_v.at[0])
    pltpu.sync_copy(perm_hbm.at[pl.ds(base + _HALF, _HALF)], idx_v.at[1])
    c0 = pltpu.async_copy(x_hbm.at[idx_v.at[0]], rows0, sem)
    c1 = pltpu.async_copy(x_hbm.at[idx_v.at[1]], rows1, sem)
    c0.wait()
    c1.wait()
    pltpu.sync_copy(rows0, out_hbm.at[pl.ds(base, _HALF)])
    pltpu.sync_copy(rows1, out_hbm.at[pl.ds(base + _HALF, _HALF)])


@functools.cache
def _sc_mesh():
    return plsc.VectorSubcoreMesh(core_axis_name="c", subcore_axis_name="s")


@functools.cache
def _dispatch_call():
    return functools.partial(
        pl.kernel,
        mesh=_sc_mesh(),
        out_type=jax.ShapeDtypeStruct((S, D), jnp.float32),
        scratch_types=[
            pltpu.VMEM((2, _HALF), jnp.int32),
            pltpu.VMEM((_HALF, D), jnp.float32),
            pltpu.VMEM((_HALF, D), jnp.float32),
            pltpu.SemaphoreType.DMA,
        ],
    )(_dispatch_body)


_FT = 768


def _mlp_body(x_ref, fc_ref, pj_ref, out_ref):
    f = pl.program_id(1)
    xb = x_ref[...]
    h = jnp.dot(xb, fc_ref[0], preferred_element_type=jnp.float32)
    g = h * 0.5 * (1.0 + lax.erf(h * 0.7071067811865476))
    contrib = jnp.dot(g, pj_ref[0], preferred_element_type=jnp.float32)

    @pl.when(f == 0)
    def _():
        out_ref[...] = contrib

    @pl.when(f != 0)
    def _():
        out_ref[...] += contrib


def _mlp(xe, c_fc, c_proj):
    return pl.pallas_call(
        _mlp_body,
        grid=(E, H // _FT),
        in_specs=[
            pl.BlockSpec((CAP, D), lambda e, f: (e, 0)),
            pl.BlockSpec((1, D, _FT), lambda e, f: (e, 0, f)),
            pl.BlockSpec((1, _FT, D), lambda e, f: (e, f, 0)),
        ],
        out_specs=pl.BlockSpec((CAP, D), lambda e, f: (e, 0)),
        out_shape=jax.ShapeDtypeStruct((S, D), jnp.float32),
        compiler_params=pltpu.CompilerParams(
            dimension_semantics=("arbitrary", "arbitrary")),
    )(xe, c_fc, c_proj)


_TOK_PER_W = T // NW


def _combine_body(eo_hbm, slot_hbm, w_hbm, out_hbm, idx_v, w_v, r0, r1, sem):
    wid = lax.axis_index("s") * NC + lax.axis_index("c")
    base = wid * _TOK_PER_W
    pltpu.sync_copy(slot_hbm.at[pl.ds(base, _TOK_PER_W)], idx_v.at[0])
    pltpu.sync_copy(slot_hbm.at[pl.ds(T + base, _TOK_PER_W)], idx_v.at[1])
    pltpu.sync_copy(w_hbm.at[pl.ds(base, _TOK_PER_W), :], w_v.at[0])
    pltpu.sync_copy(w_hbm.at[pl.ds(T + base, _TOK_PER_W), :], w_v.at[1])
    c0 = pltpu.async_copy(eo_hbm.at[idx_v.at[0]], r0, sem)
    c1 = pltpu.async_copy(eo_hbm.at[idx_v.at[1]], r1, sem)
    c0.wait()
    c1.wait()

    def tok_body(t, _):
        w0 = w_v[0, t, :]
        w1 = w_v[1, t, :]

        def vec_body(v, _):
            sl = pl.ds(v * 16, 16)
            r0[t, sl] = r0[t, sl] * w0 + r1[t, sl] * w1
            return 0

        lax.fori_loop(0, D // 16, vec_body, 0)
        return 0

    lax.fori_loop(0, _TOK_PER_W, tok_body, 0)
    pltpu.sync_copy(r0, out_hbm.at[pl.ds(base, _TOK_PER_W)])


@functools.cache
def _combine_call():
    return functools.partial(
        pl.kernel,
        mesh=_sc_mesh(),
        out_type=jax.ShapeDtypeStruct((T, D), jnp.float32),
        scratch_types=[
            pltpu.VMEM((2, _TOK_PER_W), jnp.int32),
            pltpu.VMEM((2, _TOK_PER_W, 16), jnp.float32),
            pltpu.VMEM((_TOK_PER_W, D), jnp.float32),
            pltpu.VMEM((_TOK_PER_W, D), jnp.float32),
            pltpu.SemaphoreType.DMA,
        ],
    )(_combine_body)


def kernel(x, w_g, c_fc, c_proj):
    x2 = x.reshape(T, D)
    perm, slot_c, w = _router(x2, w_g)
    xe = _dispatch_call()(x2, perm.reshape(S))
    eo = _mlp(xe, c_fc, c_proj)
    out = _combine_call()(eo, slot_c.reshape(NENT), w)
    return out.reshape(x.shape)


# speedup vs baseline: 1.5678x; 1.0008x over previous
import functools

import jax
import jax.numpy as jnp
from jax import lax
from jax.experimental import pallas as pl
from jax.experimental.pallas import tpu as pltpu
from jax.experimental.pallas import tpu_sc as plsc

D = 768
H = 4 * D
E = 8
TOPK = 2
T = 2048
CAP = 640
S = E * CAP
NENT = TOPK * T
NC = 2
NS = 16
NW = NC * NS

_BLK = 128
_NBLK = NENT // _BLK
_SBLK = S // 128


def _router_body(x_ref, wg_ref, perm_ref, slotc_ref, w_ref, slotd_ref, tok_ref):
    xv = x_ref[...]
    wg = wg_ref[...]
    logits = jnp.dot(xv, wg, preferred_element_type=jnp.float32)

    ii = lax.broadcasted_iota(jnp.int32, (T, E), 1)
    l0 = jnp.max(logits, axis=1, keepdims=True)
    e0 = jnp.min(jnp.where(logits == l0, ii, E), axis=1, keepdims=True)
    m1 = jnp.where(ii == e0, -jnp.inf, logits)
    l1 = jnp.max(m1, axis=1, keepdims=True)
    e1 = jnp.min(jnp.where(m1 == l1, ii, E), axis=1, keepdims=True)
    z = jnp.exp(l1 - l0)
    p0 = 1.0 / (1.0 + z)
    p1 = z / (1.0 + z)

    r = lax.broadcasted_iota(jnp.int32, (_BLK, _BLK), 0)
    c = lax.broadcasted_iota(jnp.int32, (_BLK, _BLK), 1)
    tri = (r >= c).astype(jnp.float32)
    blk_ii = lax.broadcasted_iota(jnp.int32, (_BLK, E), 1)
    row_iota = lax.broadcasted_iota(jnp.int32, (_BLK, 1), 0)

    carry = jnp.zeros((1, E), jnp.float32)
    for b in range(_NBLK):
        if b < _NBLK // 2:
            eb = lax.slice(e0, (b * _BLK, 0), ((b + 1) * _BLK, 1))
            pb = lax.slice(p0, (b * _BLK, 0), ((b + 1) * _BLK, 1))
            tokb = b * _BLK + row_iota
        else:
            b2 = b - _NBLK // 2
            eb = lax.slice(e1, (b2 * _BLK, 0), ((b2 + 1) * _BLK, 1))
            pb = lax.slice(p1, (b2 * _BLK, 0), ((b2 + 1) * _BLK, 1))
            tokb = b2 * _BLK + row_iota
        mb = (blk_ii == eb).astype(jnp.float32)
        inc = jnp.dot(tri, mb, preferred_element_type=jnp.float32) + carry
        carry = carry + jnp.sum(mb, axis=0, keepdims=True)
        rank = jnp.sum(mb * inc, axis=1, keepdims=True).astype(jnp.int32) - 1
        kept = rank < CAP
        slot_d = jnp.where(kept, eb * CAP + rank, -1)
        slot_c = jnp.where(kept, eb * CAP + rank, eb * CAP + (CAP - 1))
        w_out = jnp.where(kept, pb, 0.0)
        sl = pl.ds(b * _BLK, _BLK)
        slotd_ref[sl, :] = slot_d
        slotc_ref[sl, :] = slot_c
        w_ref[sl, :] = jnp.broadcast_to(w_out, (_BLK, 16))
        tok_ref[sl, :] = tokb.astype(jnp.float32)

    lane = lax.broadcasted_iota(jnp.int32, (1, 128), 1)

    def sblk_body(b, _):
        svals = b * 128 + lane

        def chunk(ic, acc):
            sd = slotd_ref[pl.ds(ic * 512, 512), :]
            tk = tok_ref[pl.ds(ic * 512, 512), :]
            m = sd == svals
            return acc + jnp.sum(jnp.where(m, tk, 0.0), axis=0, keepdims=True)

        acc = lax.fori_loop(0, NENT // 512, chunk, jnp.zeros((1, 128), jnp.float32))
        perm_ref[pl.ds(b, 1), :] = acc.astype(jnp.int32)
        return 0

    lax.fori_loop(0, _SBLK, sblk_body, 0)


def _router(x2, w_g):
    return pl.pallas_call(
        _router_body,
        out_shape=[
            jax.ShapeDtypeStruct((_SBLK, 128), jnp.int32),
            jax.ShapeDtypeStruct((NENT, 1), jnp.int32),
            jax.ShapeDtypeStruct((NENT, 16), jnp.float32),
        ],
        scratch_shapes=[
            pltpu.VMEM((NENT, 1), jnp.int32),
            pltpu.VMEM((NENT, 1), jnp.float32),
        ],
    )(x2, w_g)


_ROWS_PER_W = S // NW
_HALF = _ROWS_PER_W // 2


def _dispatch_body(x_hbm, perm_hbm, out_hbm, idx_v, rows0, rows1, sem):
    wid = lax.axis_index("s") * NC + lax.axis_index("c")
    base = wid * _ROWS_PER_W
    pltpu.sync_copy(perm_hbm.at[pl.ds(base, _HALF)], idx_v.at[0])
    pltpu.sync_copy(perm_hbm.at[pl.ds(base + _HALF, _HALF)], idx_v.at[1])
    c0 = pltpu.async_copy(x_hbm.at[idx_v.at[0]], rows0, sem)
    c1 = pltpu.async_copy(x_hbm.at[idx_v.at[1]], rows1, sem)
    c0.wait()
    c1.wait()
    pltpu.sync_copy(rows0, out_hbm.at[pl.ds(base, _HALF)])
    pltpu.sync_copy(rows1, out_hbm.at[pl.ds(base + _HALF, _HALF)])


@functools.cache
def _sc_mesh():
    return plsc.VectorSubcoreMesh(core_axis_name="c", subcore_axis_name="s")


@functools.cache
def _dispatch_call():
    return functools.partial(
        pl.kernel,
        mesh=_sc_mesh(),
        out_type=jax.ShapeDtypeStruct((S, D), jnp.float32),
        scratch_types=[
            pltpu.VMEM((2, _HALF), jnp.int32),
            pltpu.VMEM((_HALF, D), jnp.float32),
            pltpu.VMEM((_HALF, D), jnp.float32),
            pltpu.SemaphoreType.DMA,
        ],
    )(_dispatch_body)


_FT = 768


def _mlp_body(x_ref, fc_ref, pj_ref, out_ref):
    f = pl.program_id(1)
    xb = x_ref[...].astype(jnp.bfloat16)
    fc = fc_ref[0].astype(jnp.bfloat16)
    h = jnp.dot(xb, fc, preferred_element_type=jnp.float32)
    g = h * 0.5 * (1.0 + lax.erf(h * 0.7071067811865476))
    pj = pj_ref[0].astype(jnp.bfloat16)
    contrib = jnp.dot(g.astype(jnp.bfloat16), pj, preferred_element_type=jnp.float32)

    @pl.when(f == 0)
    def _():
        out_ref[...] = contrib

    @pl.when(f != 0)
    def _():
        out_ref[...] += contrib


def _mlp(xe, c_fc, c_proj):
    return pl.pallas_call(
        _mlp_body,
        grid=(E, H // _FT),
        in_specs=[
            pl.BlockSpec((CAP, D), lambda e, f: (e, 0)),
            pl.BlockSpec((1, D, _FT), lambda e, f: (e, 0, f)),
            pl.BlockSpec((1, _FT, D), lambda e, f: (e, f, 0)),
        ],
        out_specs=pl.BlockSpec((CAP, D), lambda e, f: (e, 0)),
        out_shape=jax.ShapeDtypeStruct((S, D), jnp.float32),
        compiler_params=pltpu.CompilerParams(
            dimension_semantics=("arbitrary", "arbitrary")),
    )(xe, c_fc, c_proj)


_TOK_PER_W = T // NW


def _combine_body(eo_hbm, slot_hbm, w_hbm, out_hbm, idx_v, w_v, r0, r1, sem):
    wid = lax.axis_index("s") * NC + lax.axis_index("c")
    base = wid * _TOK_PER_W
    pltpu.sync_copy(slot_hbm.at[pl.ds(base, _TOK_PER_W)], idx_v.at[0])
    pltpu.sync_copy(slot_hbm.at[pl.ds(T + base, _TOK_PER_W)], idx_v.at[1])
    pltpu.sync_copy(w_hbm.at[pl.ds(base, _TOK_PER_W), :], w_v.at[0])
    pltpu.sync_copy(w_hbm.at[pl.ds(T + base, _TOK_PER_W), :], w_v.at[1])
    c0 = pltpu.async_copy(eo_hbm.at[idx_v.at[0]], r0, sem)
    c1 = pltpu.async_copy(eo_hbm.at[idx_v.at[1]], r1, sem)
    c0.wait()
    c1.wait()

    def tok_body(t, _):
        w0 = w_v[0, t, :]
        w1 = w_v[1, t, :]

        def vec_body(v, _):
            sl = pl.ds(v * 16, 16)
            r0[t, sl] = r0[t, sl] * w0 + r1[t, sl] * w1
            return 0

        lax.fori_loop(0, D // 16, vec_body, 0)
        return 0

    lax.fori_loop(0, _TOK_PER_W, tok_body, 0)
    pltpu.sync_copy(r0, out_hbm.at[pl.ds(base, _TOK_PER_W)])


@functools.cache
def _combine_call():
    return functools.partial(
        pl.kernel,
        mesh=_sc_mesh(),
        out_type=jax.ShapeDtypeStruct((T, D), jnp.float32),
        scratch_types=[
            pltpu.VMEM((2, _TOK_PER_W), jnp.int32),
            pltpu.VMEM((2, _TOK_PER_W, 16), jnp.float32),
            pltpu.VMEM((_TOK_PER_W, D), jnp.float32),
            pltpu.VMEM((_TOK_PER_W, D), jnp.float32),
            pltpu.SemaphoreType.DMA,
        ],
    )(_combine_body)


def kernel(x, w_g, c_fc, c_proj):
    x2 = x.reshape(T, D)
    perm, slot_c, w = _router(x2, w_g)
    xe = _dispatch_call()(x2, perm.reshape(S))
    eo = _mlp(xe, c_fc, c_proj)
    out = _combine_call()(eo, slot_c.reshape(NENT), w)
    return out.reshape(x.shape)


# speedup vs baseline: 2.6193x; 1.6707x over previous
import functools

import jax
import jax.numpy as jnp
from jax import lax
from jax.experimental import pallas as pl
from jax.experimental.pallas import tpu as pltpu
from jax.experimental.pallas import tpu_sc as plsc

D = 768
H = 4 * D
E = 8
TOPK = 2
T = 2048
CAP = 640
S = E * CAP
NENT = TOPK * T
NC = 2
NS = 16
NW = NC * NS

_BLK = 512
_NBLK = NENT // _BLK
_ENT_PER_W = NENT // NW
S_FULL = S + _ENT_PER_W


def _router_body(x_ref, wg_ref, slotc_ref, slots_ref, w_ref):
    xv = x_ref[...]
    wg = wg_ref[...]
    logits = jnp.dot(xv, wg, preferred_element_type=jnp.float32)

    ii = lax.broadcasted_iota(jnp.int32, (T, E), 1)
    l0 = jnp.max(logits, axis=1, keepdims=True)
    e0 = jnp.min(jnp.where(logits == l0, ii, E), axis=1, keepdims=True)
    m1 = jnp.where(ii == e0, -jnp.inf, logits)
    l1 = jnp.max(m1, axis=1, keepdims=True)
    e1 = jnp.min(jnp.where(m1 == l1, ii, E), axis=1, keepdims=True)
    z = jnp.exp(l1 - l0)
    p0 = 1.0 / (1.0 + z)
    p1 = z / (1.0 + z)

    e_full = jnp.concatenate([e0, e1], axis=0)
    p_full = jnp.concatenate([p0, p1], axis=0)

    ii_ent = lax.broadcasted_iota(jnp.int32, (NENT, E), 1)
    M = (ii_ent == e_full).astype(jnp.float32)

    r = lax.broadcasted_iota(jnp.int32, (_BLK, _BLK), 0)
    c = lax.broadcasted_iota(jnp.int32, (_BLK, _BLK), 1)
    tri = (r >= c).astype(jnp.float32)
    carry = jnp.zeros((1, E), jnp.float32)
    ranks = []
    for b in range(_NBLK):
        mb = lax.slice(M, (b * _BLK, 0), ((b + 1) * _BLK, E))
        inc = jnp.dot(tri, mb, preferred_element_type=jnp.float32) + carry
        carry = carry + jnp.sum(mb, axis=0, keepdims=True)
        ranks.append(jnp.round(jnp.sum(mb * inc, axis=1, keepdims=True)) - 1)
    rank = jnp.concatenate(ranks, axis=0).astype(jnp.int32)

    kept = rank < CAP
    slotc_ref[...] = jnp.where(kept, e_full * CAP + rank, e_full * CAP + (CAP - 1))
    ent_i = lax.broadcasted_iota(jnp.int32, (NENT, 1), 0)
    slots_ref[...] = jnp.where(kept, e_full * CAP + rank,
                               S + (ent_i & (_ENT_PER_W - 1)))
    w_ref[...] = jnp.broadcast_to(jnp.where(kept, p_full, 0.0), (NENT, 16))


def _router(x2, w_g):
    return pl.pallas_call(
        _router_body,
        out_shape=[
            jax.ShapeDtypeStruct((NENT, 1), jnp.int32),
            jax.ShapeDtypeStruct((NENT, 1), jnp.int32),
            jax.ShapeDtypeStruct((NENT, 16), jnp.float32),
        ],
    )(x2, w_g)


def _dispatch_body(x_hbm, slot_hbm, out_hbm, idx_v, rows, sem):
    wid = lax.axis_index("s") * NC + lax.axis_index("c")
    base = wid * _ENT_PER_W
    tok_base = lax.rem(base, T)
    pltpu.sync_copy(slot_hbm.at[pl.ds(base, _ENT_PER_W)], idx_v.at[0])
    pltpu.sync_copy(x_hbm.at[pl.ds(tok_base, _ENT_PER_W)], rows)
    pltpu.async_copy(rows, out_hbm.at[idx_v.at[0]], sem).wait()


@functools.cache
def _sc_mesh():
    return plsc.VectorSubcoreMesh(core_axis_name="c", subcore_axis_name="s")


@functools.cache
def _dispatch_call():
    return functools.partial(
        pl.kernel,
        mesh=_sc_mesh(),
        out_type=jax.ShapeDtypeStruct((S_FULL, D), jnp.float32),
        scratch_types=[
            pltpu.VMEM((1, _ENT_PER_W), jnp.int32),
            pltpu.VMEM((_ENT_PER_W, D), jnp.float32),
            pltpu.SemaphoreType.DMA,
        ],
    )(_dispatch_body)


_FT = 768


def _mlp_body(x_ref, fc_ref, pj_ref, out_ref):
    f = pl.program_id(1)
    xb = x_ref[...].astype(jnp.bfloat16)
    fc = fc_ref[0].astype(jnp.bfloat16)
    h = jnp.dot(xb, fc, preferred_element_type=jnp.float32)
    g = h * 0.5 * (1.0 + lax.erf(h * 0.7071067811865476))
    pj = pj_ref[0].astype(jnp.bfloat16)
    contrib = jnp.dot(g.astype(jnp.bfloat16), pj, preferred_element_type=jnp.float32)

    @pl.when(f == 0)
    def _():
        out_ref[...] = contrib

    @pl.when(f != 0)
    def _():
        out_ref[...] += contrib


def _mlp(xe, c_fc, c_proj):
    return pl.pallas_call(
        _mlp_body,
        grid=(E, H // _FT),
        in_specs=[
            pl.BlockSpec((CAP, D), lambda e, f: (e, 0)),
            pl.BlockSpec((1, D, _FT), lambda e, f: (e, 0, f)),
            pl.BlockSpec((1, _FT, D), lambda e, f: (e, f, 0)),
        ],
        out_specs=pl.BlockSpec((CAP, D), lambda e, f: (e, 0)),
        out_shape=jax.ShapeDtypeStruct((S, D), jnp.float32),
        compiler_params=pltpu.CompilerParams(
            dimension_semantics=("arbitrary", "arbitrary")),
    )(xe, c_fc, c_proj)


_TOK_PER_W = T // NW


def _combine_body(eo_hbm, slot_hbm, w_hbm, out_hbm, idx_v, w_v, r0, r1, sem):
    wid = lax.axis_index("s") * NC + lax.axis_index("c")
    base = wid * _TOK_PER_W
    pltpu.sync_copy(slot_hbm.at[pl.ds(base, _TOK_PER_W)], idx_v.at[0])
    pltpu.sync_copy(slot_hbm.at[pl.ds(T + base, _TOK_PER_W)], idx_v.at[1])
    pltpu.sync_copy(w_hbm.at[pl.ds(base, _TOK_PER_W), :], w_v.at[0])
    pltpu.sync_copy(w_hbm.at[pl.ds(T + base, _TOK_PER_W), :], w_v.at[1])
    c0 = pltpu.async_copy(eo_hbm.at[idx_v.at[0]], r0, sem)
    c1 = pltpu.async_copy(eo_hbm.at[idx_v.at[1]], r1, sem)
    c0.wait()
    c1.wait()

    def tok_body(t, _):
        w0 = w_v[0, t, :]
        w1 = w_v[1, t, :]

        zero = jnp.zeros((16,), jnp.float32)

        def vec_body(v, _):
            sl = pl.ds(v * 16, 16)
            a = jnp.where(w0 > 0.0, r0[t, sl] * w0, zero)
            b = jnp.where(w1 > 0.0, r1[t, sl] * w1, zero)
            r0[t, sl] = a + b
            return 0

        lax.fori_loop(0, D // 16, vec_body, 0)
        return 0

    lax.fori_loop(0, _TOK_PER_W, tok_body, 0)
    pltpu.sync_copy(r0, out_hbm.at[pl.ds(base, _TOK_PER_W)])


@functools.cache
def _combine_call():
    return functools.partial(
        pl.kernel,
        mesh=_sc_mesh(),
        out_type=jax.ShapeDtypeStruct((T, D), jnp.float32),
        scratch_types=[
            pltpu.VMEM((2, _TOK_PER_W), jnp.int32),
            pltpu.VMEM((2, _TOK_PER_W, 16), jnp.float32),
            pltpu.VMEM((_TOK_PER_W, D), jnp.float32),
            pltpu.VMEM((_TOK_PER_W, D), jnp.float32),
            pltpu.SemaphoreType.DMA,
        ],
    )(_combine_body)


def kernel(x, w_g, c_fc, c_proj):
    x2 = x.reshape(T, D)
    slot_c, slot_s, w = _router(x2, w_g)
    xe = _dispatch_call()(x2, slot_s.reshape(NENT))
    eo = _mlp(xe, c_fc, c_proj)
    out = _combine_call()(eo, slot_c.reshape(NENT), w)
    return out.reshape(x.shape)


# speedup vs baseline: 2.6888x; 1.0265x over previous
import functools

import jax
import jax.numpy as jnp
from jax import lax
from jax.experimental import pallas as pl
from jax.experimental.pallas import tpu as pltpu
from jax.experimental.pallas import tpu_sc as plsc

D = 768
H = 4 * D
E = 8
TOPK = 2
T = 2048
CAP = 640
S = E * CAP
NENT = TOPK * T
NC = 2
NS = 16
NW = NC * NS

_BLK = 512
_NBLK = NENT // _BLK
_ENT_PER_W = NENT // NW
S_FULL = S + _ENT_PER_W


def _router_body(x_ref, wg_ref, slotc_ref, slots_ref, w_ref):
    xv = x_ref[...]
    wg = wg_ref[...]
    logits = jnp.dot(xv, wg, preferred_element_type=jnp.float32)

    ii = lax.broadcasted_iota(jnp.int32, (T, E), 1)
    l0 = jnp.max(logits, axis=1, keepdims=True)
    e0 = jnp.min(jnp.where(logits == l0, ii, E), axis=1, keepdims=True)
    m1 = jnp.where(ii == e0, -jnp.inf, logits)
    l1 = jnp.max(m1, axis=1, keepdims=True)
    e1 = jnp.min(jnp.where(m1 == l1, ii, E), axis=1, keepdims=True)
    z = jnp.exp(l1 - l0)
    p0 = 1.0 / (1.0 + z)
    p1 = z / (1.0 + z)

    e_full = jnp.concatenate([e0, e1], axis=0)
    p_full = jnp.concatenate([p0, p1], axis=0)

    r = lax.broadcasted_iota(jnp.int32, (_BLK, _BLK), 0)
    c = lax.broadcasted_iota(jnp.int32, (_BLK, _BLK), 1)
    tri = (r >= c).astype(jnp.float32)
    ii8 = lax.broadcasted_iota(jnp.int32, (_BLK, E), 1)
    mbs = []
    for b in range(_NBLK):
        eb = lax.slice(e_full, (b * _BLK, 0), ((b + 1) * _BLK, 1))
        mbs.append((ii8 == eb).astype(jnp.float32))
    m2 = jnp.concatenate(mbs, axis=1)
    pres = [jnp.zeros((1, E), jnp.float32)]
    for b in range(1, _NBLK):
        pres.append(pres[-1] + jnp.sum(mbs[b - 1], axis=0, keepdims=True))
    pre2 = jnp.concatenate(pres, axis=1)
    inc_all = jnp.dot(tri, m2, preferred_element_type=jnp.float32) + pre2
    sel = m2 * inc_all
    ranks = []
    for b in range(_NBLK):
        sb = lax.slice(sel, (0, b * E), (_BLK, (b + 1) * E))
        ranks.append(jnp.round(jnp.sum(sb, axis=1, keepdims=True)) - 1)
    rank = jnp.concatenate(ranks, axis=0).astype(jnp.int32)

    kept = rank < CAP
    slotc_ref[...] = jnp.where(kept, e_full * CAP + rank, e_full * CAP + (CAP - 1))
    ent_i = lax.broadcasted_iota(jnp.int32, (NENT, 1), 0)
    slots_ref[...] = jnp.where(kept, e_full * CAP + rank,
                               S + (ent_i & (_ENT_PER_W - 1)))
    w_ref[...] = jnp.broadcast_to(jnp.where(kept, p_full, 0.0), (NENT, 16))


def _router(x2, w_g):
    return pl.pallas_call(
        _router_body,
        out_shape=[
            jax.ShapeDtypeStruct((NENT, 1), jnp.int32),
            jax.ShapeDtypeStruct((NENT, 1), jnp.int32),
            jax.ShapeDtypeStruct((NENT, 16), jnp.float32),
        ],
    )(x2, w_g)


def _dispatch_body(x_hbm, slot_hbm, out_hbm, idx_v, rows, sem, semi):
    wid = lax.axis_index("s") * NC + lax.axis_index("c")
    base = wid * _ENT_PER_W
    tok_base = lax.rem(base, T)
    ci = pltpu.async_copy(slot_hbm.at[pl.ds(base, _ENT_PER_W)], idx_v.at[0], semi)
    cr = pltpu.async_copy(x_hbm.at[pl.ds(tok_base, _ENT_PER_W)], rows, sem)
    ci.wait()
    cr.wait()
    pltpu.async_copy(rows, out_hbm.at[idx_v.at[0]], sem).wait()


@functools.cache
def _sc_mesh():
    return plsc.VectorSubcoreMesh(core_axis_name="c", subcore_axis_name="s")


@functools.cache
def _dispatch_call():
    return functools.partial(
        pl.kernel,
        mesh=_sc_mesh(),
        out_type=jax.ShapeDtypeStruct((S_FULL, D), jnp.float32),
        scratch_types=[
            pltpu.VMEM((1, _ENT_PER_W), jnp.int32),
            pltpu.VMEM((_ENT_PER_W, D), jnp.float32),
            pltpu.SemaphoreType.DMA,
            pltpu.SemaphoreType.DMA,
        ],
    )(_dispatch_body)


_FT = 768


def _mlp_body(x_ref, fc_ref, pj_ref, out_ref):
    f = pl.program_id(1)
    xb = x_ref[...].astype(jnp.bfloat16)
    fc = fc_ref[0].astype(jnp.bfloat16)
    h = jnp.dot(xb, fc, preferred_element_type=jnp.float32)
    g = h * 0.5 * (1.0 + lax.erf(h * 0.7071067811865476))
    pj = pj_ref[0].astype(jnp.bfloat16)
    contrib = jnp.dot(g.astype(jnp.bfloat16), pj, preferred_element_type=jnp.float32)

    @pl.when(f == 0)
    def _():
        out_ref[...] = contrib

    @pl.when(f != 0)
    def _():
        out_ref[...] += contrib


def _mlp(xe, c_fc, c_proj):
    return pl.pallas_call(
        _mlp_body,
        grid=(E, H // _FT),
        in_specs=[
            pl.BlockSpec((CAP, D), lambda e, f: (e, 0)),
            pl.BlockSpec((1, D, _FT), lambda e, f: (e, 0, f)),
            pl.BlockSpec((1, _FT, D), lambda e, f: (e, f, 0)),
        ],
        out_specs=pl.BlockSpec((CAP, D), lambda e, f: (e, 0)),
        out_shape=jax.ShapeDtypeStruct((S, D), jnp.float32),
        compiler_params=pltpu.CompilerParams(
            dimension_semantics=("arbitrary", "arbitrary")),
    )(xe, c_fc, c_proj)


_TOK_PER_W = T // NW


_WV = _TOK_PER_W // 2


def _combine_body(eo_hbm, slot_hbm, w_hbm, out_hbm,
                  idx_v, w_v, r0a, r1a, r0b, r1b, sa, sb, swr):
    wid = lax.axis_index("s") * NC + lax.axis_index("c")
    base = wid * _TOK_PER_W
    pltpu.sync_copy(slot_hbm.at[pl.ds(base, _WV)], idx_v.at[0])
    pltpu.sync_copy(slot_hbm.at[pl.ds(base + _WV, _WV)], idx_v.at[1])
    pltpu.sync_copy(slot_hbm.at[pl.ds(T + base, _WV)], idx_v.at[2])
    pltpu.sync_copy(slot_hbm.at[pl.ds(T + base + _WV, _WV)], idx_v.at[3])
    pltpu.sync_copy(w_hbm.at[pl.ds(base, _TOK_PER_W), :], w_v.at[0])
    pltpu.sync_copy(w_hbm.at[pl.ds(T + base, _TOK_PER_W), :], w_v.at[1])
    ga0 = pltpu.async_copy(eo_hbm.at[idx_v.at[0]], r0a, sa)
    ga1 = pltpu.async_copy(eo_hbm.at[idx_v.at[2]], r1a, sa)
    gb0 = pltpu.async_copy(eo_hbm.at[idx_v.at[1]], r0b, sb)
    gb1 = pltpu.async_copy(eo_hbm.at[idx_v.at[3]], r1b, sb)

    zero = jnp.zeros((16,), jnp.float32)

    def wave(rr0, rr1, toff):
        def tok_body(t, _):
            w0 = w_v[0, toff + t, :]
            w1 = w_v[1, toff + t, :]

            def vec_body(v, _):
                for u in range(4):
                    sl = pl.ds(v * 64 + u * 16, 16)
                    a = jnp.where(w0 > 0.0, rr0[t, sl] * w0, zero)
                    b = jnp.where(w1 > 0.0, rr1[t, sl] * w1, zero)
                    rr0[t, sl] = a + b
                return 0

            lax.fori_loop(0, D // 64, vec_body, 0)
            return 0

        lax.fori_loop(0, _WV, tok_body, 0)

    ga0.wait()
    ga1.wait()
    wave(r0a, r1a, 0)
    wa = pltpu.async_copy(r0a, out_hbm.at[pl.ds(base, _WV)], swr)
    gb0.wait()
    gb1.wait()
    wave(r0b, r1b, _WV)
    wb = pltpu.async_copy(r0b, out_hbm.at[pl.ds(base + _WV, _WV)], swr)
    wa.wait()
    wb.wait()


@functools.cache
def _combine_call():
    return functools.partial(
        pl.kernel,
        mesh=_sc_mesh(),
        out_type=jax.ShapeDtypeStruct((T, D), jnp.float32),
        scratch_types=[
            pltpu.VMEM((4, _WV), jnp.int32),
            pltpu.VMEM((2, _TOK_PER_W, 16), jnp.float32),
            pltpu.VMEM((_WV, D), jnp.float32),
            pltpu.VMEM((_WV, D), jnp.float32),
            pltpu.VMEM((_WV, D), jnp.float32),
            pltpu.VMEM((_WV, D), jnp.float32),
            pltpu.SemaphoreType.DMA,
            pltpu.SemaphoreType.DMA,
            pltpu.SemaphoreType.DMA,
        ],
    )(_combine_body)


def kernel(x, w_g, c_fc, c_proj):
    x2 = x.reshape(T, D)
    slot_c, slot_s, w = _router(x2, w_g)
    xe = _dispatch_call()(x2, slot_s.reshape(NENT))
    eo = _mlp(xe, c_fc, c_proj)
    eo = _mlp(xe, c_fc, c_proj)
    out = _combine_call()(eo, slot_c.reshape(NENT), w)
    return out.reshape(x.shape)
